# baseline (speedup 1.0000x reference)
import jax
import jax.numpy as jnp
from jax import lax
from jax.experimental import pallas as pl
from jax.experimental.pallas import tpu as pltpu
from jax.experimental.pallas import tpu_sc as plsc

T = 2048
H = 1024
E = 8
DFF = 768
P = T * 2
BLK = 128
MAXBLOCKS = 39
MAXROWS = MAXBLOCKS * BLK
NBID = 48
NC, NS, L = 2, 16, 16
NW = NC * NS
_MESH = dict(core_axis_name="c", subcore_axis_name="s")
TBLK = 256


def _worker_id():
  return lax.axis_index("s") * NC + lax.axis_index("c")


def _gating_body(x_ref, gw_ref, e1_ref, e2_ref, w1_ref, w2_ref):
  lg = jnp.dot(x_ref[...], gw_ref[...], preferred_element_type=jnp.float32)
  lane = lax.broadcasted_iota(jnp.int32, (TBLK, 128), 1)
  neg = jnp.where(lane < E, lg, -3e38)
  l1 = jnp.max(neg, axis=1, keepdims=True)
  e1 = jnp.min(jnp.where(neg == l1, lane, 128), axis=1, keepdims=True)
  neg2 = jnp.where(lane == e1, -3e38, neg)
  l2 = jnp.max(neg2, axis=1, keepdims=True)
  e2 = jnp.min(jnp.where(neg2 == l2, lane, 128), axis=1, keepdims=True)
  w1 = 1.0 / (1.0 + jnp.exp(l2 - l1))
  e1_ref[...] = e1
  e2_ref[...] = e2
  w1_ref[...] = w1
  w2_ref[...] = 1.0 - w1


def _gating(x, gw_pad):
  out = pl.pallas_call(
      _gating_body,
      grid=(T // TBLK,),
      in_specs=[
          pl.BlockSpec((TBLK, H), lambda m: (m, 0)),
          pl.BlockSpec((H, 128), lambda m: (0, 0)),
      ],
      out_specs=[pl.BlockSpec((TBLK, 1), lambda m: (m, 0))] * 4,
      out_shape=(
          jax.ShapeDtypeStruct((T, 1), jnp.int32),
          jax.ShapeDtypeStruct((T, 1), jnp.int32),
          jax.ShapeDtypeStruct((T, 1), jnp.float32),
          jax.ShapeDtypeStruct((T, 1), jnp.float32),
      ),
  )(x, gw_pad)
  return out


RW = 16
RPW = P // RW
RG = RPW // L


def _route_body(e1_hbm, e2_hbm, x_hbm, wflat_hbm,
                pos_hbm, bid_hbm, xs_hbm, ws_hbm,
                evw, rankw, posw, cntw, bidv, sh, tbl,
                idx_v, w_v, rows_v, sem):
  sid = lax.axis_index("s")
  cid = lax.axis_index("c")
  zero = jnp.zeros((L,), jnp.int32)
  one = jnp.full((L,), 1, jnp.int32)
  lanes = jnp.arange(L, dtype=jnp.int32)

  def shift_r(v, d):
    sh[pl.ds(L, L)] = v
    return sh[pl.ds(L - d, L)]

  def shift_l(v, d):
    sh[pl.ds(L, L)] = v
    return sh[pl.ds(L + d, L)]

  @pl.when(cid == 0)
  def _():
    sh[pl.ds(0, L)] = zero
    sh[pl.ds(2 * L, L)] = zero

    @pl.when(sid < RW // 2)
    def _():
      pltpu.sync_copy(e1_hbm.at[pl.ds(sid * RPW, RPW)], evw)

    @pl.when(sid >= RW // 2)
    def _():
      pltpu.sync_copy(e2_hbm.at[pl.ds((sid - RW // 2) * RPW, RPW)], evw)

    lcnt = [zero] * E
    for g in range(RG):
      e = evw[pl.ds(g * L, L)]
      rank = zero
      for k in range(E):
        mi = jnp.where(e == k, one, zero)
        rank = rank + mi * lcnt[k]
        lcnt[k] = lcnt[k] + mi
      rankw[pl.ds(g * L, L)] = rank

    for k in range(E):
      incl = lcnt[k]
      for d in (1, 2, 4, 8):
        incl = incl + shift_r(incl, d)
      cntw[pl.ds((E + k) * L, L)] = incl - lcnt[k]
      tot = incl
      for d in (1, 2, 4, 8):
        tot = jnp.maximum(tot, shift_l(tot, d))
      cntw[pl.ds(k * L, L)] = tot
    pltpu.sync_copy(cntw.at[pl.ds(0, E * L)],
                    tbl.at[pl.ds(sid * (E * L), E * L)])

  plsc.subcore_barrier()

  @pl.when(cid == 0)
  def _():
    pltpu.sync_copy(tbl, posw)
    widv = jnp.full((L,), 1, jnp.int32) * sid
    gtot = [zero] * E
    wpre = [zero] * E
    for w in range(RW):
      mask = jnp.where(jnp.full((L,), w, jnp.int32) < widv, one, zero)
      for k in range(E):
        v = posw[pl.ds((w * E + k) * L, L)]
        gtot[k] = gtot[k] + v
        wpre[k] = wpre[k] + v * mask

    offs, endsb = [], []
    acc = zero
    for k in range(E):
      blk = lax.shift_left(
          lax.shift_right_logical(gtot[k] + (BLK - 1), 7), 7)
      offs.append(acc)
      acc = acc + blk
      endsb.append(lax.shift_right_logical(acc, 7))

    @pl.when(sid == 0)
    def _():
      for j in range(NBID // L):
        bv = lanes + j * L
        a = zero
        for k in range(E):
          a = a + jnp.where(bv >= endsb[k], one, zero)
        bidv[pl.ds(j * L, L)] = jnp.minimum(a, E - 1)
      pltpu.sync_copy(bidv, bid_hbm)

    base = [offs[k] + wpre[k] + cntw[pl.ds((E + k) * L, L)]
            for k in range(E)]
    for g in range(RG):
      sl = pl.ds(g * L, L)
      e = evw[sl]
      p = rankw[sl]
      for k in range(E):
        p = p + jnp.where(e == k, base[k], zero)
      posw[sl] = p
    pltpu.sync_copy(posw.at[pl.ds(0, RPW)], pos_hbm.at[pl.ds(sid * RPW, RPW)])

    dchunk = 64
    for c in range(RPW // dchunk):
      b = pl.multiple_of(sid * RPW + c * dchunk, dchunk)
      tok = pl.multiple_of(jnp.bitwise_and(b, T - 1), dchunk)
      for q in range(dchunk // L):
        idx_v[pl.ds(q * L, L)] = posw[pl.ds(c * dchunk + q * L, L)]
      pltpu.sync_copy(x_hbm.at[pl.ds(tok, dchunk)], rows_v)
      pltpu.sync_copy(wflat_hbm.at[pl.ds(b, dchunk)], w_v)
      pltpu.async_copy(rows_v, xs_hbm.at[idx_v], sem).wait()
      pltpu.async_copy(w_v, ws_hbm.at[idx_v], sem).wait()


def _route(e1, e2, x, wflat):
  dchunk = 64
  return pl.kernel(
      _route_body,
      out_type=(
          jax.ShapeDtypeStruct((P,), jnp.int32),
          jax.ShapeDtypeStruct((NBID,), jnp.int32),
          jax.ShapeDtypeStruct((MAXROWS, H), jnp.float32),
          jax.ShapeDtypeStruct((MAXROWS,), jnp.float32),
      ),
      mesh=plsc.VectorSubcoreMesh(**_MESH),
      scratch_types=[
          pltpu.VMEM((RPW,), jnp.int32),
          pltpu.VMEM((RPW,), jnp.int32),
          pltpu.VMEM((RW * E * L,), jnp.int32),
          pltpu.VMEM((2 * E * L,), jnp.int32),
          pltpu.VMEM((NBID,), jnp.int32),
          pltpu.VMEM((3 * L,), jnp.int32),
          pltpu.VMEM_SHARED((RW * E * L,), jnp.int32),
          pltpu.VMEM((dchunk,), jnp.int32),
          pltpu.VMEM((dchunk,), jnp.float32),
          pltpu.VMEM((dchunk, H), jnp.float32),
          pltpu.SemaphoreType.DMA,
      ],
  )(e1, e2, x, wflat)


def _gemm_body(bid_ref, xs_ref, ws_ref, wg_ref, wu_ref, wd_ref, y_ref):
  del bid_ref
  xb = xs_ref[...]
  g = jnp.dot(xb, wg_ref[0], preferred_element_type=jnp.float32,
              precision=lax.Precision.DEFAULT)
  u = jnp.dot(xb, wu_ref[0], preferred_element_type=jnp.float32,
              precision=lax.Precision.DEFAULT)
  act = g * (1.0 / (1.0 + jnp.exp(-g))) * u
  y = jnp.dot(act, wd_ref[0], preferred_element_type=jnp.float32,
              precision=lax.Precision.DEFAULT)
  y_ref[...] = y * ws_ref[...]


def _gemm(bid, xs, ws, w_gate, w_up, w_down):
  grid_spec = pltpu.PrefetchScalarGridSpec(
      num_scalar_prefetch=1,
      grid=(MAXBLOCKS,),
      in_specs=[
          pl.BlockSpec((BLK, H), lambda m, b: (m, 0)),
          pl.BlockSpec((BLK, 1), lambda m, b: (m, 0)),
          pl.BlockSpec((1, H, DFF), lambda m, b: (b[m], 0, 0)),
          pl.BlockSpec((1, H, DFF), lambda m, b: (b[m], 0, 0)),
          pl.BlockSpec((1, DFF, H), lambda m, b: (b[m], 0, 0)),
      ],
      out_specs=pl.BlockSpec((BLK, H), lambda m, b: (m, 0)),
  )
  return pl.pallas_call(
      _gemm_body,
      grid_spec=grid_spec,
      out_shape=jax.ShapeDtypeStruct((MAXROWS, H), jnp.float32),
  )(bid, xs, ws, w_gate, w_up, w_down)


def _combine_body(y_hbm, pos_hbm, out_hbm, i1, i2, b1, b2, ob, sem):
  tpw = T // NW
  chunk = tpw // 2
  t0 = _worker_id() * tpw
  for c in range(2):
    tc0 = t0 + c * chunk
    pltpu.sync_copy(pos_hbm.at[pl.ds(tc0, chunk)], i1)
    pltpu.sync_copy(pos_hbm.at[pl.ds(T + tc0, chunk)], i2)
    pltpu.async_copy(y_hbm.at[i1], b1, sem).wait()
    pltpu.async_copy(y_hbm.at[i2], b2, sem).wait()
    for r in range(chunk):
      def cbody(cc, ci, r=r):
        off = pl.multiple_of(cc * L, L)
        ob[r, pl.ds(off, L)] = (b1[r, pl.ds(off, L)]
                                + b2[r, pl.ds(off, L)])
        return ci
      lax.fori_loop(0, H // L, cbody, 0, unroll=False)
    pltpu.sync_copy(ob, out_hbm.at[pl.ds(tc0, chunk)])


def _combine(y, pos):
  chunk = T // NW // 2
  return pl.kernel(
      _combine_body,
      out_type=jax.ShapeDtypeStruct((T, H), jnp.float32),
      mesh=plsc.VectorSubcoreMesh(**_MESH),
      scratch_types=[
          pltpu.VMEM((chunk,), jnp.int32),
          pltpu.VMEM((chunk,), jnp.int32),
          pltpu.VMEM((chunk, H), jnp.float32),
          pltpu.VMEM((chunk, H), jnp.float32),
          pltpu.VMEM((chunk, H), jnp.float32),
          pltpu.SemaphoreType.DMA,
      ],
  )(y, pos)


@jax.jit
def kernel(hidden_states, gate_w, w_gate, w_up, w_down):
  x = hidden_states.reshape(T, H)
  gw_pad = jnp.pad(gate_w.astype(jnp.float32), ((0, 0), (0, 128 - E)))
  e1c, e2c, w1c, w2c = _gating(x, gw_pad)
  e1 = e1c.reshape(T)
  e2 = e2c.reshape(T)
  wflat = jnp.concatenate([w1c.reshape(T), w2c.reshape(T)])
  pos, bid, xs, ws = _route(e1, e2, x, wflat)
  y = _gemm(bid, xs, ws.reshape(MAXROWS, 1), w_gate, w_up, w_down)
  out = _combine(y, pos)
  return out.reshape(hidden_states.shape)


# speedup vs baseline: 1.0720x; 1.0720x over previous
import jax
import jax.numpy as jnp
from jax import lax
from jax.experimental import pallas as pl
from jax.experimental.pallas import tpu as pltpu
from jax.experimental.pallas import tpu_sc as plsc

T = 2048
H = 1024
E = 8
DFF = 768
P = T * 2
BLK = 128
MAXBLOCKS = 39
MAXROWS = MAXBLOCKS * BLK
NBID = 48
NC, NS, L = 2, 16, 16
NW = NC * NS
_MESH = dict(core_axis_name="c", subcore_axis_name="s")
TBLK = 256


def _worker_id():
  return lax.axis_index("s") * NC + lax.axis_index("c")


def _gating_body(x_ref, gw_ref, e1_ref, e2_ref, w1_ref, w2_ref):
  lg = jnp.dot(x_ref[...], gw_ref[...], preferred_element_type=jnp.float32)
  lane = lax.broadcasted_iota(jnp.int32, (TBLK, 128), 1)
  neg = jnp.where(lane < E, lg, -3e38)
  l1 = jnp.max(neg, axis=1, keepdims=True)
  e1 = jnp.min(jnp.where(neg == l1, lane, 128), axis=1, keepdims=True)
  neg2 = jnp.where(lane == e1, -3e38, neg)
  l2 = jnp.max(neg2, axis=1, keepdims=True)
  e2 = jnp.min(jnp.where(neg2 == l2, lane, 128), axis=1, keepdims=True)
  w1 = 1.0 / (1.0 + jnp.exp(l2 - l1))
  e1_ref[...] = e1
  e2_ref[...] = e2
  w1_ref[...] = w1
  w2_ref[...] = 1.0 - w1


def _gating(x, gw_pad):
  out = pl.pallas_call(
      _gating_body,
      grid=(T // TBLK,),
      in_specs=[
          pl.BlockSpec((TBLK, H), lambda m: (m, 0)),
          pl.BlockSpec((H, 128), lambda m: (0, 0)),
      ],
      out_specs=[pl.BlockSpec((TBLK, 1), lambda m: (m, 0))] * 4,
      out_shape=(
          jax.ShapeDtypeStruct((T, 1), jnp.int32),
          jax.ShapeDtypeStruct((T, 1), jnp.int32),
          jax.ShapeDtypeStruct((T, 1), jnp.float32),
          jax.ShapeDtypeStruct((T, 1), jnp.float32),
      ),
  )(x, gw_pad)
  return out


RW = 16
RPW = P // RW
RG = RPW // L


def _route_body(e1_hbm, e2_hbm, pos_hbm, bid_hbm,
                evw, rankw, posw, cntw, bidv, sh, tbl):
  sid = lax.axis_index("s")
  cid = lax.axis_index("c")
  zero = jnp.zeros((L,), jnp.int32)
  one = jnp.full((L,), 1, jnp.int32)
  lanes = jnp.arange(L, dtype=jnp.int32)

  def shift_r(v, d):
    sh[pl.ds(L, L)] = v
    return sh[pl.ds(L - d, L)]

  def shift_l(v, d):
    sh[pl.ds(L, L)] = v
    return sh[pl.ds(L + d, L)]

  @pl.when(cid == 0)
  def _():
    sh[pl.ds(0, L)] = zero
    sh[pl.ds(2 * L, L)] = zero

    @pl.when(sid < RW // 2)
    def _():
      pltpu.sync_copy(e1_hbm.at[pl.ds(sid * RPW, RPW)], evw)

    @pl.when(sid >= RW // 2)
    def _():
      pltpu.sync_copy(e2_hbm.at[pl.ds((sid - RW // 2) * RPW, RPW)], evw)

    lcnt = [zero] * E
    for g in range(RG):
      e = evw[pl.ds(g * L, L)]
      rank = zero
      for k in range(E):
        mi = jnp.where(e == k, one, zero)
        rank = rank + mi * lcnt[k]
        lcnt[k] = lcnt[k] + mi
      rankw[pl.ds(g * L, L)] = rank

    for k in range(E):
      incl = lcnt[k]
      for d in (1, 2, 4, 8):
        incl = incl + shift_r(incl, d)
      cntw[pl.ds((E + k) * L, L)] = incl - lcnt[k]
      tot = incl
      for d in (1, 2, 4, 8):
        tot = jnp.maximum(tot, shift_l(tot, d))
      cntw[pl.ds(k * L, L)] = tot
    pltpu.sync_copy(cntw.at[pl.ds(0, E * L)],
                    tbl.at[pl.ds(sid * (E * L), E * L)])

  plsc.subcore_barrier()

  @pl.when(cid == 0)
  def _():
    pltpu.sync_copy(tbl, posw)
    widv = jnp.full((L,), 1, jnp.int32) * sid
    gtot = [zero] * E
    wpre = [zero] * E
    for w in range(RW):
      mask = jnp.where(jnp.full((L,), w, jnp.int32) < widv, one, zero)
      for k in range(E):
        v = posw[pl.ds((w * E + k) * L, L)]
        gtot[k] = gtot[k] + v
        wpre[k] = wpre[k] + v * mask

    offs, endsb = [], []
    acc = zero
    for k in range(E):
      blk = lax.shift_left(
          lax.shift_right_logical(gtot[k] + (BLK - 1), 7), 7)
      offs.append(acc)
      acc = acc + blk
      endsb.append(lax.shift_right_logical(acc, 7))

    @pl.when(sid == 0)
    def _():
      for j in range(NBID // L):
        bv = lanes + j * L
        a = zero
        for k in range(E):
          a = a + jnp.where(bv >= endsb[k], one, zero)
        bidv[pl.ds(j * L, L)] = jnp.minimum(a, E - 1)
      pltpu.sync_copy(bidv, bid_hbm)

    base = [offs[k] + wpre[k] + cntw[pl.ds((E + k) * L, L)]
            for k in range(E)]
    for g in range(RG):
      sl = pl.ds(g * L, L)
      e = evw[sl]
      p = rankw[sl]
      for k in range(E):
        p = p + jnp.where(e == k, base[k], zero)
      posw[sl] = p
    pltpu.sync_copy(posw.at[pl.ds(0, RPW)], pos_hbm.at[pl.ds(sid * RPW, RPW)])


def _route(e1, e2):
  return pl.kernel(
      _route_body,
      out_type=(
          jax.ShapeDtypeStruct((P,), jnp.int32),
          jax.ShapeDtypeStruct((NBID,), jnp.int32),
      ),
      mesh=plsc.VectorSubcoreMesh(**_MESH),
      scratch_types=[
          pltpu.VMEM((RPW,), jnp.int32),
          pltpu.VMEM((RPW,), jnp.int32),
          pltpu.VMEM((RW * E * L,), jnp.int32),
          pltpu.VMEM((2 * E * L,), jnp.int32),
          pltpu.VMEM((NBID,), jnp.int32),
          pltpu.VMEM((3 * L,), jnp.int32),
          pltpu.VMEM_SHARED((RW * E * L,), jnp.int32),
      ],
  )(e1, e2)


DCH = 32
DNC = (P // NW) // DCH


def _dispatch_body(x_hbm, pos_hbm, wflat_hbm, xs_hbm, ws_hbm,
                   idx_all, w_v, i0, i1, r0, r1,
                   sin0, sin1, sout0, sout1):
  ppw = P // NW
  base = pl.multiple_of(_worker_id() * ppw, ppw)
  tokbase = pl.multiple_of(jnp.bitwise_and(base, T - 1), ppw)
  pltpu.sync_copy(pos_hbm.at[pl.ds(base, ppw)], idx_all)
  pltpu.sync_copy(wflat_hbm.at[pl.ds(base, ppw)], w_v)
  idxb = [i0, i1]
  rows = [r0, r1]
  sin = [sin0, sin1]
  sout = [sout0, sout1]

  def fill_idx(buf, c):
    for q in range(DCH // L):
      idxb[buf][pl.ds(q * L, L)] = idx_all[pl.ds(c * DCH + q * L, L)]

  def start_in(c):
    return pltpu.async_copy(
        x_hbm.at[pl.ds(tokbase + c * DCH, DCH)], rows[c % 2], sin[c % 2])

  def start_out(c):
    return pltpu.async_copy(
        rows[c % 2], xs_hbm.at[idxb[c % 2]], sout[c % 2])

  fill_idx(0, 0)
  fill_idx(1, 1)
  in0 = start_in(0)
  in1 = start_in(1)
  in0.wait()
  out0 = start_out(0)
  in1.wait()
  out1 = start_out(1)
  out0.wait()
  fill_idx(0, 2)
  in2 = start_in(2)
  out1.wait()
  fill_idx(1, 3)
  in3 = start_in(3)
  in2.wait()
  out2 = start_out(2)
  in3.wait()
  out3 = start_out(3)
  wcp = pltpu.async_copy(w_v, ws_hbm.at[idx_all], sin0)
  out2.wait()
  out3.wait()
  wcp.wait()


def _dispatch(x, pos, wflat):
  ppw = P // NW
  return pl.kernel(
      _dispatch_body,
      out_type=(
          jax.ShapeDtypeStruct((MAXROWS, H), jnp.float32),
          jax.ShapeDtypeStruct((MAXROWS,), jnp.float32),
      ),
      mesh=plsc.VectorSubcoreMesh(**_MESH),
      scratch_types=[
          pltpu.VMEM((ppw,), jnp.int32),
          pltpu.VMEM((ppw,), jnp.float32),
          pltpu.VMEM((DCH,), jnp.int32),
          pltpu.VMEM((DCH,), jnp.int32),
          pltpu.VMEM((DCH, H), jnp.float32),
          pltpu.VMEM((DCH, H), jnp.float32),
          pltpu.SemaphoreType.DMA,
          pltpu.SemaphoreType.DMA,
          pltpu.SemaphoreType.DMA,
          pltpu.SemaphoreType.DMA,
      ],
  )(x, pos, wflat)


def _gemm_body(bid_ref, xs_ref, ws_ref, wg_ref, wu_ref, wd_ref, y_ref):
  del bid_ref
  xb = xs_ref[...]
  g = jnp.dot(xb, wg_ref[0], preferred_element_type=jnp.float32,
              precision=lax.Precision.DEFAULT)
  u = jnp.dot(xb, wu_ref[0], preferred_element_type=jnp.float32,
              precision=lax.Precision.DEFAULT)
  act = g * (1.0 / (1.0 + jnp.exp(-g))) * u
  y = jnp.dot(act, wd_ref[0], preferred_element_type=jnp.float32,
              precision=lax.Precision.DEFAULT)
  y_ref[...] = y * ws_ref[...]


def _gemm(bid, xs, ws, w_gate, w_up, w_down):
  grid_spec = pltpu.PrefetchScalarGridSpec(
      num_scalar_prefetch=1,
      grid=(MAXBLOCKS,),
      in_specs=[
          pl.BlockSpec((BLK, H), lambda m, b: (m, 0)),
          pl.BlockSpec((BLK, 1), lambda m, b: (m, 0)),
          pl.BlockSpec((1, H, DFF), lambda m, b: (b[m], 0, 0)),
          pl.BlockSpec((1, H, DFF), lambda m, b: (b[m], 0, 0)),
          pl.BlockSpec((1, DFF, H), lambda m, b: (b[m], 0, 0)),
      ],
      out_specs=pl.BlockSpec((BLK, H), lambda m, b: (m, 0)),
  )
  return pl.pallas_call(
      _gemm_body,
      grid_spec=grid_spec,
      out_shape=jax.ShapeDtypeStruct((MAXROWS, H), jnp.float32),
  )(bid, xs, ws, w_gate, w_up, w_down)


def _combine_body(y_hbm, pos_hbm, out_hbm, i1, i2, b1, b2, ob, sem):
  tpw = T // NW
  chunk = tpw // 2
  t0 = _worker_id() * tpw
  for c in range(2):
    tc0 = t0 + c * chunk
    pltpu.sync_copy(pos_hbm.at[pl.ds(tc0, chunk)], i1)
    pltpu.sync_copy(pos_hbm.at[pl.ds(T + tc0, chunk)], i2)
    pltpu.async_copy(y_hbm.at[i1], b1, sem).wait()
    pltpu.async_copy(y_hbm.at[i2], b2, sem).wait()
    for r in range(chunk):
      def cbody(cc, ci, r=r):
        off = pl.multiple_of(cc * L, L)
        ob[r, pl.ds(off, L)] = (b1[r, pl.ds(off, L)]
                                + b2[r, pl.ds(off, L)])
        return ci
      lax.fori_loop(0, H // L, cbody, 0, unroll=False)
    pltpu.sync_copy(ob, out_hbm.at[pl.ds(tc0, chunk)])


def _combine(y, pos):
  chunk = T // NW // 2
  return pl.kernel(
      _combine_body,
      out_type=jax.ShapeDtypeStruct((T, H), jnp.float32),
      mesh=plsc.VectorSubcoreMesh(**_MESH),
      scratch_types=[
          pltpu.VMEM((chunk,), jnp.int32),
          pltpu.VMEM((chunk,), jnp.int32),
          pltpu.VMEM((chunk, H), jnp.float32),
          pltpu.VMEM((chunk, H), jnp.float32),
          pltpu.VMEM((chunk, H), jnp.float32),
          pltpu.SemaphoreType.DMA,
      ],
  )(y, pos)


@jax.jit
def kernel(hidden_states, gate_w, w_gate, w_up, w_down):
  x = hidden_states.reshape(T, H)
  gw_pad = jnp.pad(gate_w.astype(jnp.float32), ((0, 0), (0, 128 - E)))
  e1c, e2c, w1c, w2c = _gating(x, gw_pad)
  e1 = e1c.reshape(T)
  e2 = e2c.reshape(T)
  wflat = jnp.concatenate([w1c.reshape(T), w2c.reshape(T)])
  pos, bid = _route(e1, e2)
  xs, ws = _dispatch(x, pos, wflat)
  y = _gemm(bid, xs, ws.reshape(MAXROWS, 1), w_gate, w_up, w_down)
  out = _combine(y, pos)
  return out.reshape(hidden_states.shape)


# speedup vs baseline: 1.0837x; 1.0109x over previous
import jax
import jax.numpy as jnp
from jax import lax
from jax.experimental import pallas as pl
from jax.experimental.pallas import tpu as pltpu
from jax.experimental.pallas import tpu_sc as plsc

T = 2048
H = 1024
E = 8
DFF = 768
P = T * 2
BLK = 128
MAXBLOCKS = 39
MAXROWS = MAXBLOCKS * BLK
NBID = 48
NC, NS, L = 2, 16, 16
NW = NC * NS
_MESH = dict(core_axis_name="c", subcore_axis_name="s")
TBLK = 256


def _worker_id():
  return lax.axis_index("s") * NC + lax.axis_index("c")


def _gating_body(x_ref, gw_ref, e1_ref, e2_ref, w1_ref, w2_ref):
  lg = jnp.dot(x_ref[...], gw_ref[...], preferred_element_type=jnp.float32)
  lane = lax.broadcasted_iota(jnp.int32, (TBLK, 128), 1)
  neg = jnp.where(lane < E, lg, -3e38)
  l1 = jnp.max(neg, axis=1, keepdims=True)
  e1 = jnp.min(jnp.where(neg == l1, lane, 128), axis=1, keepdims=True)
  neg2 = jnp.where(lane == e1, -3e38, neg)
  l2 = jnp.max(neg2, axis=1, keepdims=True)
  e2 = jnp.min(jnp.where(neg2 == l2, lane, 128), axis=1, keepdims=True)
  w1 = 1.0 / (1.0 + jnp.exp(l2 - l1))
  e1_ref[...] = e1
  e2_ref[...] = e2
  w1_ref[...] = w1
  w2_ref[...] = 1.0 - w1


def _gating(x, gw_pad):
  out = pl.pallas_call(
      _gating_body,
      grid=(T // TBLK,),
      in_specs=[
          pl.BlockSpec((TBLK, H), lambda m: (m, 0)),
          pl.BlockSpec((H, 128), lambda m: (0, 0)),
      ],
      out_specs=[pl.BlockSpec((TBLK, 1), lambda m: (m, 0))] * 4,
      out_shape=(
          jax.ShapeDtypeStruct((T, 1), jnp.int32),
          jax.ShapeDtypeStruct((T, 1), jnp.int32),
          jax.ShapeDtypeStruct((T, 1), jnp.float32),
          jax.ShapeDtypeStruct((T, 1), jnp.float32),
      ),
  )(x, gw_pad)
  return out


RW = 16
RPW = P // RW
RG = RPW // L


def _route_body(e1_hbm, e2_hbm, pos_hbm, bid_hbm,
                evw, rankw, posw, cntw, bidv, sh, tbl):
  sid = lax.axis_index("s")
  cid = lax.axis_index("c")
  zero = jnp.zeros((L,), jnp.int32)
  one = jnp.full((L,), 1, jnp.int32)
  lanes = jnp.arange(L, dtype=jnp.int32)

  def shift_r(v, d):
    sh[pl.ds(L, L)] = v
    return sh[pl.ds(L - d, L)]

  def shift_l(v, d):
    sh[pl.ds(L, L)] = v
    return sh[pl.ds(L + d, L)]

  @pl.when(cid == 0)
  def _():
    sh[pl.ds(0, L)] = zero
    sh[pl.ds(2 * L, L)] = zero

    @pl.when(sid < RW // 2)
    def _():
      pltpu.sync_copy(e1_hbm.at[pl.ds(sid * RPW, RPW)], evw)

    @pl.when(sid >= RW // 2)
    def _():
      pltpu.sync_copy(e2_hbm.at[pl.ds((sid - RW // 2) * RPW, RPW)], evw)

    lcnt = [zero] * E
    for g in range(RG):
      e = evw[pl.ds(g * L, L)]
      rank = zero
      for k in range(E):
        mi = jnp.where(e == k, one, zero)
        rank = rank + mi * lcnt[k]
        lcnt[k] = lcnt[k] + mi
      rankw[pl.ds(g * L, L)] = rank

    for k in range(E):
      incl = lcnt[k]
      for d in (1, 2, 4, 8):
        incl = incl + shift_r(incl, d)
      cntw[pl.ds((E + k) * L, L)] = incl - lcnt[k]
      tot = incl
      for d in (1, 2, 4, 8):
        tot = jnp.maximum(tot, shift_l(tot, d))
      cntw[pl.ds(k * L, L)] = tot
    pltpu.sync_copy(cntw.at[pl.ds(0, E * L)],
                    tbl.at[pl.ds(sid * (E * L), E * L)])

  plsc.subcore_barrier()

  @pl.when(cid == 0)
  def _():
    pltpu.sync_copy(tbl, posw)
    widv = jnp.full((L,), 1, jnp.int32) * sid
    gtot = [zero] * E
    wpre = [zero] * E
    for w in range(RW):
      mask = jnp.where(jnp.full((L,), w, jnp.int32) < widv, one, zero)
      for k in range(E):
        v = posw[pl.ds((w * E + k) * L, L)]
        gtot[k] = gtot[k] + v
        wpre[k] = wpre[k] + v * mask

    offs, endsb = [], []
    acc = zero
    for k in range(E):
      blk = lax.shift_left(
          lax.shift_right_logical(gtot[k] + (BLK - 1), 7), 7)
      offs.append(acc)
      acc = acc + blk
      endsb.append(lax.shift_right_logical(acc, 7))

    @pl.when(sid == 0)
    def _():
      for j in range(NBID // L):
        bv = lanes + j * L
        a = zero
        for k in range(E):
          a = a + jnp.where(bv >= endsb[k], one, zero)
        bidv[pl.ds(j * L, L)] = jnp.minimum(a, E - 1)
      pltpu.sync_copy(bidv, bid_hbm)

    base = [offs[k] + wpre[k] + cntw[pl.ds((E + k) * L, L)]
            for k in range(E)]
    for g in range(RG):
      sl = pl.ds(g * L, L)
      e = evw[sl]
      p = rankw[sl]
      for k in range(E):
        p = p + jnp.where(e == k, base[k], zero)
      posw[sl] = p
    pltpu.sync_copy(posw.at[pl.ds(0, RPW)], pos_hbm.at[pl.ds(sid * RPW, RPW)])


def _route(e1, e2):
  return pl.kernel(
      _route_body,
      out_type=(
          jax.ShapeDtypeStruct((P,), jnp.int32),
          jax.ShapeDtypeStruct((NBID,), jnp.int32),
      ),
      mesh=plsc.VectorSubcoreMesh(**_MESH),
      scratch_types=[
          pltpu.VMEM((RPW,), jnp.int32),
          pltpu.VMEM((RPW,), jnp.int32),
          pltpu.VMEM((RW * E * L,), jnp.int32),
          pltpu.VMEM((2 * E * L,), jnp.int32),
          pltpu.VMEM((NBID,), jnp.int32),
          pltpu.VMEM((3 * L,), jnp.int32),
          pltpu.VMEM_SHARED((RW * E * L,), jnp.int32),
      ],
  )(e1, e2)


def _dispatch_body(x_hbm, pos_hbm, wflat_hbm, xs_hbm, ws_hbm,
                   idx_v, w_v, rows_v, sem):
  ppw = P // NW
  chunk = ppw // 2
  base = _worker_id() * ppw
  for c in range(2):
    b = pl.multiple_of(base + c * chunk, chunk)
    tok = pl.multiple_of(jnp.bitwise_and(b, T - 1), chunk)
    pltpu.sync_copy(pos_hbm.at[pl.ds(b, chunk)], idx_v)
    pltpu.sync_copy(x_hbm.at[pl.ds(tok, chunk)], rows_v)
    pltpu.sync_copy(wflat_hbm.at[pl.ds(b, chunk)], w_v)
    pltpu.async_copy(rows_v, xs_hbm.at[idx_v], sem).wait()
    pltpu.async_copy(w_v, ws_hbm.at[idx_v], sem).wait()


def _dispatch(x, pos, wflat):
  chunk = P // NW // 2
  return pl.kernel(
      _dispatch_body,
      out_type=(
          jax.ShapeDtypeStruct((MAXROWS, H), jnp.float32),
          jax.ShapeDtypeStruct((MAXROWS,), jnp.float32),
      ),
      mesh=plsc.VectorSubcoreMesh(**_MESH),
      scratch_types=[
          pltpu.VMEM((chunk,), jnp.int32),
          pltpu.VMEM((chunk,), jnp.float32),
          pltpu.VMEM((chunk, H), jnp.float32),
          pltpu.SemaphoreType.DMA,
      ],
  )(x, pos, wflat)


def _gemm_body(bid_ref, xs_ref, ws_ref, wg_ref, wu_ref, wd_ref, y_ref):
  del bid_ref
  xb = xs_ref[...]
  g = jnp.dot(xb, wg_ref[0], preferred_element_type=jnp.float32,
              precision=lax.Precision.DEFAULT)
  u = jnp.dot(xb, wu_ref[0], preferred_element_type=jnp.float32,
              precision=lax.Precision.DEFAULT)
  act = g * (1.0 / (1.0 + jnp.exp(-g))) * u
  y = jnp.dot(act, wd_ref[0], preferred_element_type=jnp.float32,
              precision=lax.Precision.DEFAULT)
  y_ref[...] = y * ws_ref[...]


def _gemm(bid, xs, ws, w_gate, w_up, w_down):
  grid_spec = pltpu.PrefetchScalarGridSpec(
      num_scalar_prefetch=1,
      grid=(MAXBLOCKS,),
      in_specs=[
          pl.BlockSpec((BLK, H), lambda m, b: (m, 0)),
          pl.BlockSpec((BLK, 1), lambda m, b: (m, 0)),
          pl.BlockSpec((1, H, DFF), lambda m, b: (b[m], 0, 0)),
          pl.BlockSpec((1, H, DFF), lambda m, b: (b[m], 0, 0)),
          pl.BlockSpec((1, DFF, H), lambda m, b: (b[m], 0, 0)),
      ],
      out_specs=pl.BlockSpec((BLK, H), lambda m, b: (m, 0)),
  )
  return pl.pallas_call(
      _gemm_body,
      grid_spec=grid_spec,
      out_shape=jax.ShapeDtypeStruct((MAXROWS, H), jnp.float32),
  )(bid, xs, ws, w_gate, w_up, w_down)


def _combine_body(y_hbm, pos_hbm, out_hbm, i1, i2, b1, b2, ob, sem):
  tpw = T // NW
  chunk = tpw // 2
  t0 = _worker_id() * tpw
  for c in range(2):
    tc0 = t0 + c * chunk
    pltpu.sync_copy(pos_hbm.at[pl.ds(tc0, chunk)], i1)
    pltpu.sync_copy(pos_hbm.at[pl.ds(T + tc0, chunk)], i2)
    pltpu.async_copy(y_hbm.at[i1], b1, sem).wait()
    pltpu.async_copy(y_hbm.at[i2], b2, sem).wait()
    for r in range(chunk):
      def cbody(cc, ci, r=r):
        off = pl.multiple_of(cc * L, L)
        ob[r, pl.ds(off, L)] = (b1[r, pl.ds(off, L)]
                                + b2[r, pl.ds(off, L)])
        return ci
      lax.fori_loop(0, H // L, cbody, 0, unroll=False)
    pltpu.sync_copy(ob, out_hbm.at[pl.ds(tc0, chunk)])


def _combine(y, pos):
  chunk = T // NW // 2
  return pl.kernel(
      _combine_body,
      out_type=jax.ShapeDtypeStruct((T, H), jnp.float32),
      mesh=plsc.VectorSubcoreMesh(**_MESH),
      scratch_types=[
          pltpu.VMEM((chunk,), jnp.int32),
          pltpu.VMEM((chunk,), jnp.int32),
          pltpu.VMEM((chunk, H), jnp.float32),
          pltpu.VMEM((chunk, H), jnp.float32),
          pltpu.VMEM((chunk, H), jnp.float32),
          pltpu.SemaphoreType.DMA,
      ],
  )(y, pos)


@jax.jit
def kernel(hidden_states, gate_w, w_gate, w_up, w_down):
  x = hidden_states.reshape(T, H)
  gw_pad = jnp.pad(gate_w.astype(jnp.float32), ((0, 0), (0, 128 - E)))
  e1c, e2c, w1c, w2c = _gating(x, gw_pad)
  e1 = e1c.reshape(T)
  e2 = e2c.reshape(T)
  wflat = jnp.concatenate([w1c.reshape(T), w2c.reshape(T)])
  pos, bid = _route(e1, e2)
  xs, ws = _dispatch(x, pos, wflat)
  y = _gemm(bid, xs, ws.reshape(MAXROWS, 1), w_gate, w_up, w_down)
  out = _combine(y, pos)
  return out.reshape(hidden_states.shape)


# speedup vs baseline: 1.1610x; 1.0713x over previous
import jax
import jax.numpy as jnp
from jax import lax
from jax.experimental import pallas as pl
from jax.experimental.pallas import tpu as pltpu
from jax.experimental.pallas import tpu_sc as plsc

T = 2048
H = 1024
E = 8
DFF = 768
P = T * 2
BLK = 128
MAXBLOCKS = 39
MAXROWS = MAXBLOCKS * BLK
NBID = 48
NC, NS, L = 2, 16, 16
NW = NC * NS
_MESH = dict(core_axis_name="c", subcore_axis_name="s")
TBLK = 256


def _worker_id():
  return lax.axis_index("s") * NC + lax.axis_index("c")


def _gating_body(x_ref, gw_ref, e1_ref, e2_ref, w1_ref, w2_ref):
  lg = jnp.dot(x_ref[...], gw_ref[...], preferred_element_type=jnp.float32)
  lane = lax.broadcasted_iota(jnp.int32, (TBLK, 128), 1)
  neg = jnp.where(lane < E, lg, -3e38)
  l1 = jnp.max(neg, axis=1, keepdims=True)
  e1 = jnp.min(jnp.where(neg == l1, lane, 128), axis=1, keepdims=True)
  neg2 = jnp.where(lane == e1, -3e38, neg)
  l2 = jnp.max(neg2, axis=1, keepdims=True)
  e2 = jnp.min(jnp.where(neg2 == l2, lane, 128), axis=1, keepdims=True)
  w1 = 1.0 / (1.0 + jnp.exp(l2 - l1))
  e1_ref[...] = e1
  e2_ref[...] = e2
  w1_ref[...] = w1
  w2_ref[...] = 1.0 - w1


def _gating(x, gw_pad):
  out = pl.pallas_call(
      _gating_body,
      grid=(T // TBLK,),
      in_specs=[
          pl.BlockSpec((TBLK, H), lambda m: (m, 0)),
          pl.BlockSpec((H, 128), lambda m: (0, 0)),
      ],
      out_specs=[pl.BlockSpec((TBLK, 1), lambda m: (m, 0))] * 4,
      out_shape=(
          jax.ShapeDtypeStruct((T, 1), jnp.int32),
          jax.ShapeDtypeStruct((T, 1), jnp.int32),
          jax.ShapeDtypeStruct((T, 1), jnp.float32),
          jax.ShapeDtypeStruct((T, 1), jnp.float32),
      ),
  )(x, gw_pad)
  return out


RW = 16
RPW = P // RW
RG = RPW // L


def _route_body(e1_hbm, e2_hbm, pos_hbm, bid_hbm,
                evw, rankw, posw, cntw, bidv, sh, tbl):
  sid = lax.axis_index("s")
  cid = lax.axis_index("c")
  zero = jnp.zeros((L,), jnp.int32)
  one = jnp.full((L,), 1, jnp.int32)
  lanes = jnp.arange(L, dtype=jnp.int32)

  def shift_r(v, d):
    sh[pl.ds(L, L)] = v
    return sh[pl.ds(L - d, L)]

  def shift_l(v, d):
    sh[pl.ds(L, L)] = v
    return sh[pl.ds(L + d, L)]

  @pl.when(cid == 0)
  def _():
    sh[pl.ds(0, L)] = zero
    sh[pl.ds(2 * L, L)] = zero

    @pl.when(sid < RW // 2)
    def _():
      pltpu.sync_copy(e1_hbm.at[pl.ds(sid * RPW, RPW)], evw)

    @pl.when(sid >= RW // 2)
    def _():
      pltpu.sync_copy(e2_hbm.at[pl.ds((sid - RW // 2) * RPW, RPW)], evw)

    lcnt = [zero] * E
    for g in range(RG):
      e = evw[pl.ds(g * L, L)]
      rank = zero
      for k in range(E):
        mi = jnp.where(e == k, one, zero)
        rank = rank + mi * lcnt[k]
        lcnt[k] = lcnt[k] + mi
      rankw[pl.ds(g * L, L)] = rank

    for k in range(E):
      incl = lcnt[k]
      for d in (1, 2, 4, 8):
        incl = incl + shift_r(incl, d)
      cntw[pl.ds((E + k) * L, L)] = incl - lcnt[k]
      tot = incl
      for d in (1, 2, 4, 8):
        tot = jnp.maximum(tot, shift_l(tot, d))
      cntw[pl.ds(k * L, L)] = tot
    pltpu.sync_copy(cntw.at[pl.ds(0, E * L)],
                    tbl.at[pl.ds(sid * (E * L), E * L)])

  plsc.subcore_barrier()

  @pl.when(cid == 0)
  def _():
    pltpu.sync_copy(tbl, posw)
    widv = jnp.full((L,), 1, jnp.int32) * sid
    gtot = [zero] * E
    wpre = [zero] * E
    for w in range(RW):
      mask = jnp.where(jnp.full((L,), w, jnp.int32) < widv, one, zero)
      for k in range(E):
        v = posw[pl.ds((w * E + k) * L, L)]
        gtot[k] = gtot[k] + v
        wpre[k] = wpre[k] + v * mask

    offs, endsb = [], []
    acc = zero
    for k in range(E):
      blk = lax.shift_left(
          lax.shift_right_logical(gtot[k] + (BLK - 1), 7), 7)
      offs.append(acc)
      acc = acc + blk
      endsb.append(lax.shift_right_logical(acc, 7))

    @pl.when(sid == 0)
    def _():
      for j in range(NBID // L):
        bv = lanes + j * L
        a = zero
        for k in range(E):
          a = a + jnp.where(bv >= endsb[k], one, zero)
        bidv[pl.ds(j * L, L)] = jnp.minimum(a, E - 1)
      pltpu.sync_copy(bidv, bid_hbm)

    base = [offs[k] + wpre[k] + cntw[pl.ds((E + k) * L, L)]
            for k in range(E)]
    for g in range(RG):
      sl = pl.ds(g * L, L)
      e = evw[sl]
      p = rankw[sl]
      for k in range(E):
        p = p + jnp.where(e == k, base[k], zero)
      posw[sl] = p
    pltpu.sync_copy(posw.at[pl.ds(0, RPW)], pos_hbm.at[pl.ds(sid * RPW, RPW)])


def _route(e1, e2):
  return pl.kernel(
      _route_body,
      out_type=(
          jax.ShapeDtypeStruct((P,), jnp.int32),
          jax.ShapeDtypeStruct((NBID,), jnp.int32),
      ),
      mesh=plsc.VectorSubcoreMesh(**_MESH),
      scratch_types=[
          pltpu.VMEM((RPW,), jnp.int32),
          pltpu.VMEM((RPW,), jnp.int32),
          pltpu.VMEM((RW * E * L,), jnp.int32),
          pltpu.VMEM((2 * E * L,), jnp.int32),
          pltpu.VMEM((NBID,), jnp.int32),
          pltpu.VMEM((3 * L,), jnp.int32),
          pltpu.VMEM_SHARED((RW * E * L,), jnp.int32),
      ],
  )(e1, e2)


def _dispatch_body(x_hbm, pos_hbm, wflat_hbm, xs_hbm, ws_hbm,
                   idx_v, w_v, rows_v, sem):
  ppw = P // NW
  chunk = ppw // 2
  base = _worker_id() * ppw
  for c in range(2):
    b = pl.multiple_of(base + c * chunk, chunk)
    tok = pl.multiple_of(jnp.bitwise_and(b, T - 1), chunk)
    pltpu.sync_copy(pos_hbm.at[pl.ds(b, chunk)], idx_v)
    pltpu.sync_copy(x_hbm.at[pl.ds(tok, chunk)], rows_v)
    pltpu.sync_copy(wflat_hbm.at[pl.ds(b, chunk)], w_v)
    pltpu.async_copy(rows_v, xs_hbm.at[idx_v], sem).wait()
    pltpu.async_copy(w_v, ws_hbm.at[idx_v], sem).wait()


def _dispatch(x, pos, wflat):
  chunk = P // NW // 2
  return pl.kernel(
      _dispatch_body,
      out_type=(
          jax.ShapeDtypeStruct((MAXROWS, H), jnp.float32),
          jax.ShapeDtypeStruct((MAXROWS,), jnp.float32),
      ),
      mesh=plsc.VectorSubcoreMesh(**_MESH),
      scratch_types=[
          pltpu.VMEM((chunk,), jnp.int32),
          pltpu.VMEM((chunk,), jnp.float32),
          pltpu.VMEM((chunk, H), jnp.float32),
          pltpu.SemaphoreType.DMA,
      ],
  )(x, pos, wflat)


def _gemm_body(bid_ref, xs_ref, ws_ref, wg_ref, wu_ref, wd_ref, y_ref):
  del bid_ref
  xb = xs_ref[...]
  g = jnp.dot(xb, wg_ref[0], preferred_element_type=jnp.float32,
              precision=lax.Precision.DEFAULT)
  u = jnp.dot(xb, wu_ref[0], preferred_element_type=jnp.float32,
              precision=lax.Precision.DEFAULT)
  act = g * (1.0 / (1.0 + jnp.exp(-g))) * u
  y = jnp.dot(act, wd_ref[0], preferred_element_type=jnp.float32,
              precision=lax.Precision.DEFAULT)
  y_ref[...] = y * ws_ref[...]


def _gemm(bid, xs, ws, w_gate, w_up, w_down):
  grid_spec = pltpu.PrefetchScalarGridSpec(
      num_scalar_prefetch=1,
      grid=(MAXBLOCKS,),
      in_specs=[
          pl.BlockSpec((BLK, H), lambda m, b: (m, 0)),
          pl.BlockSpec((BLK, 1), lambda m, b: (m, 0)),
          pl.BlockSpec((1, H, DFF), lambda m, b: (b[m], 0, 0)),
          pl.BlockSpec((1, H, DFF), lambda m, b: (b[m], 0, 0)),
          pl.BlockSpec((1, DFF, H), lambda m, b: (b[m], 0, 0)),
      ],
      out_specs=pl.BlockSpec((BLK, H), lambda m, b: (m, 0)),
  )
  return pl.pallas_call(
      _gemm_body,
      grid_spec=grid_spec,
      out_shape=jax.ShapeDtypeStruct((MAXROWS, H), jnp.float32),
  )(bid, xs, ws, w_gate, w_up, w_down)


CCH = 16
CNC = (T // NW) // CCH


def _combine_body(y_hbm, pos_hbm, out_hbm,
                  i1a, i1b, i2a, i2b, b1a, b1b, b2a, b2b, sg0, sg1):
  t0 = _worker_id() * (T // NW)
  i1 = [i1a, i1b]
  i2 = [i2a, i2b]
  b1 = [b1a, b1b]
  b2 = [b2a, b2b]
  sg = [sg0, sg1]

  def start_chunk(c):
    buf = c % 2
    tc0 = t0 + c * CCH
    pltpu.sync_copy(pos_hbm.at[pl.ds(tc0, CCH)], i1[buf])
    pltpu.sync_copy(pos_hbm.at[pl.ds(T + tc0, CCH)], i2[buf])
    g1 = pltpu.async_copy(y_hbm.at[i1[buf]], b1[buf], sg[buf])
    g2 = pltpu.async_copy(y_hbm.at[i2[buf]], b2[buf], sg[buf])
    return g1, g2

  g = [None] * CNC
  g[0] = start_chunk(0)
  g[1] = start_chunk(1)
  for c in range(CNC):
    buf = c % 2
    g[c][0].wait()
    g[c][1].wait()

    def cbody(cc, ci, buf=buf):
      off = pl.multiple_of(cc * L, L)
      for r in range(CCH):
        b1[buf][r, pl.ds(off, L)] = (b1[buf][r, pl.ds(off, L)]
                                     + b2[buf][r, pl.ds(off, L)])
      return ci

    lax.fori_loop(0, H // L, cbody, 0, unroll=False)
    pltpu.sync_copy(b1[buf], out_hbm.at[pl.ds(t0 + c * CCH, CCH)])
    if c + 2 < CNC:
      g[c + 2] = start_chunk(c + 2)


def _combine(y, pos):
  return pl.kernel(
      _combine_body,
      out_type=jax.ShapeDtypeStruct((T, H), jnp.float32),
      mesh=plsc.VectorSubcoreMesh(**_MESH),
      scratch_types=[
          pltpu.VMEM((CCH,), jnp.int32),
          pltpu.VMEM((CCH,), jnp.int32),
          pltpu.VMEM((CCH,), jnp.int32),
          pltpu.VMEM((CCH,), jnp.int32),
          pltpu.VMEM((CCH, H), jnp.float32),
          pltpu.VMEM((CCH, H), jnp.float32),
          pltpu.VMEM((CCH, H), jnp.float32),
          pltpu.VMEM((CCH, H), jnp.float32),
          pltpu.SemaphoreType.DMA,
          pltpu.SemaphoreType.DMA,
      ],
  )(y, pos)


@jax.jit
def kernel(hidden_states, gate_w, w_gate, w_up, w_down):
  x = hidden_states.reshape(T, H)
  gw_pad = jnp.pad(gate_w.astype(jnp.float32), ((0, 0), (0, 128 - E)))
  e1c, e2c, w1c, w2c = _gating(x, gw_pad)
  e1 = e1c.reshape(T)
  e2 = e2c.reshape(T)
  wflat = jnp.concatenate([w1c.reshape(T), w2c.reshape(T)])
  pos, bid = _route(e1, e2)
  xs, ws = _dispatch(x, pos, wflat)
  y = _gemm(bid, xs, ws.reshape(MAXROWS, 1), w_gate, w_up, w_down)
  out = _combine(y, pos)
  return out.reshape(hidden_states.shape)
